# baseline (speedup 1.0000x reference)
import functools

import jax
import jax.numpy as jnp
from jax import lax
from jax.experimental import pallas as pl
from jax.experimental.pallas import tpu as pltpu
from jax.experimental.pallas import tpu_sc as plsc

EMB_DIM = 1024
SEQ_LEN = 8192

_NC = 2
_NS = 16
_NW = _NC * _NS
_B_PER_W = SEQ_LEN // _NW
_CHUNK = 32
_N_CHUNKS = _B_PER_W // _CHUNK
_NBUF = 3


def _sc_gather(table, idx):
    mesh = plsc.VectorSubcoreMesh(core_axis_name="c", subcore_axis_name="s")

    @functools.partial(
        pl.kernel,
        mesh=mesh,
        out_type=jax.ShapeDtypeStruct((SEQ_LEN, EMB_DIM), jnp.float32),
        scratch_types=[
            pltpu.VMEM((_N_CHUNKS, _CHUNK), jnp.int32),
            pltpu.VMEM((16,), jnp.int32),
            *[pltpu.VMEM((_CHUNK, EMB_DIM), jnp.float32) for _ in range(_NBUF)],
            *[pltpu.SemaphoreType.DMA for _ in range(2 * _NBUF)],
        ],
    )
    def k(table_hbm, idx_hbm, out_hbm, idx_v, tail_v, *scratch):
        bufs = scratch[:_NBUF]
        gsems = scratch[_NBUF:2 * _NBUF]
        ssems = scratch[2 * _NBUF:]
        wid = lax.axis_index("s") * _NC + lax.axis_index("c")
        base = wid * _B_PER_W
        pltpu.sync_copy(idx_hbm.at[wid], idx_v)

        pltpu.sync_copy(
            idx_hbm.at[wid, _N_CHUNKS - 1, pl.ds(_CHUNK - 16, 16)], tail_v)
        last = tail_v[...][15]

        @pl.when(last == base + _B_PER_W - 1)
        def _():
            pltpu.sync_copy(
                table_hbm.at[pl.ds(base, _B_PER_W)],
                out_hbm.at[pl.ds(base, _B_PER_W)])

        @pl.when(last != base + _B_PER_W - 1)
        def _():
            _slow_path(table_hbm, out_hbm, idx_v, bufs, gsems, ssems, base)

    return k(table, idx)


def _slow_path(table_hbm, out_hbm, idx_v, bufs, gsems, ssems, base):
        def start_gather(g):
            return pltpu.async_copy(
                table_hbm.at[idx_v.at[g]], bufs[g % _NBUF], gsems[g % _NBUF])

        gh, sh = {}, {}
        for g in range(min(_NBUF - 1, _N_CHUNKS)):
            gh[g] = start_gather(g)
        for j in range(_N_CHUNKS):
            g = j + _NBUF - 1
            if g < _N_CHUNKS:
                if g - _NBUF >= 0:
                    sh[g - _NBUF].wait()
                gh[g] = start_gather(g)
            gh[j].wait()
            sh[j] = pltpu.async_copy(
                bufs[j % _NBUF],
                out_hbm.at[pl.ds(base + j * _CHUNK, _CHUNK)],
                ssems[j % _NBUF])
        for j in range(max(0, _N_CHUNKS - _NBUF), _N_CHUNKS):
            sh[j].wait()


def kernel(seq_len, pos_emb):
    positions = jnp.arange(0, SEQ_LEN, dtype=jnp.int32)
    positions = jnp.minimum(positions, jnp.asarray(seq_len, dtype=jnp.int32) - 1)
    idx = positions.reshape(_NW, _N_CHUNKS, _CHUNK)
    return _sc_gather(pos_emb, idx)


# speedup vs baseline: 23.9879x; 23.9879x over previous
import functools

import jax
import jax.numpy as jnp
from jax import lax
from jax.experimental import pallas as pl
from jax.experimental.pallas import tpu as pltpu
from jax.experimental.pallas import tpu_sc as plsc

EMB_DIM = 1024
SEQ_LEN = 8192

_NC = 2
_NS = 16
_NW = _NC * _NS
_B_PER_W = SEQ_LEN // _NW
_CHUNK = 32
_N_CHUNKS = _B_PER_W // _CHUNK
_NBUF = 3


def _sc_gather(table, idx):
    mesh = plsc.VectorSubcoreMesh(core_axis_name="c", subcore_axis_name="s")

    @functools.partial(
        pl.kernel,
        mesh=mesh,
        out_type=jax.ShapeDtypeStruct((SEQ_LEN, EMB_DIM), jnp.float32),
        scratch_types=[
            pltpu.VMEM((_N_CHUNKS, _CHUNK), jnp.int32),
            *[pltpu.VMEM((_CHUNK, EMB_DIM), jnp.float32) for _ in range(_NBUF)],
            *[pltpu.SemaphoreType.DMA for _ in range(2 * _NBUF)],
        ],
    )
    def k(table_hbm, idx_hbm, out_hbm, idx_v, *scratch):
        bufs = scratch[:_NBUF]
        gsems = scratch[_NBUF:2 * _NBUF]
        ssems = scratch[2 * _NBUF:]
        wid = lax.axis_index("s") * _NC + lax.axis_index("c")
        base = wid * _B_PER_W
        pltpu.sync_copy(idx_hbm.at[wid], idx_v)

        def start_gather(g):
            return pltpu.async_copy(
                table_hbm.at[idx_v.at[g]], bufs[g % _NBUF], gsems[g % _NBUF])

        gh, sh = {}, {}
        for g in range(min(_NBUF - 1, _N_CHUNKS)):
            gh[g] = start_gather(g)
        for j in range(_N_CHUNKS):
            g = j + _NBUF - 1
            if g < _N_CHUNKS:
                if g - _NBUF >= 0:
                    sh[g - _NBUF].wait()
                gh[g] = start_gather(g)
            gh[j].wait()
            sh[j] = pltpu.async_copy(
                bufs[j % _NBUF],
                out_hbm.at[pl.ds(base + j * _CHUNK, _CHUNK)],
                ssems[j % _NBUF])
        for j in range(max(0, _N_CHUNKS - _NBUF), _N_CHUNKS):
            sh[j].wait()

    return k(table, idx)


def kernel(seq_len, pos_emb):
    positions = jnp.arange(0, SEQ_LEN, dtype=jnp.int32)
    positions = jnp.minimum(positions, jnp.asarray(seq_len, dtype=jnp.int32) - 1)
    idx = positions.reshape(_NW, _N_CHUNKS, _CHUNK)
    return _sc_gather(pos_emb, idx)


# speedup vs baseline: 24.3086x; 1.0134x over previous
import functools

import jax
import jax.numpy as jnp
from jax import lax
from jax.experimental import pallas as pl
from jax.experimental.pallas import tpu as pltpu
from jax.experimental.pallas import tpu_sc as plsc

EMB_DIM = 1024
SEQ_LEN = 8192

_NC = 2
_NS = 16
_NW = _NC * _NS
_B_PER_W = SEQ_LEN // _NW
_CHUNK = 16
_N_CHUNKS = _B_PER_W // _CHUNK
_NBUF = 6


def _sc_gather(table, idx):
    mesh = plsc.VectorSubcoreMesh(core_axis_name="c", subcore_axis_name="s")

    @functools.partial(
        pl.kernel,
        mesh=mesh,
        out_type=jax.ShapeDtypeStruct((SEQ_LEN, EMB_DIM), jnp.float32),
        scratch_types=[
            pltpu.VMEM((_N_CHUNKS, _CHUNK), jnp.int32),
            *[pltpu.VMEM((_CHUNK, EMB_DIM), jnp.float32) for _ in range(_NBUF)],
            *[pltpu.SemaphoreType.DMA for _ in range(2 * _NBUF)],
        ],
    )
    def k(table_hbm, idx_hbm, out_hbm, idx_v, *scratch):
        bufs = scratch[:_NBUF]
        gsems = scratch[_NBUF:2 * _NBUF]
        ssems = scratch[2 * _NBUF:]
        wid = lax.axis_index("s") * _NC + lax.axis_index("c")
        base = wid * _B_PER_W
        pltpu.sync_copy(idx_hbm.at[wid], idx_v)

        def start_gather(g):
            return pltpu.async_copy(
                table_hbm.at[idx_v.at[g]], bufs[g % _NBUF], gsems[g % _NBUF])

        def start_scatter(j):
            return pltpu.async_copy(
                bufs[j % _NBUF],
                out_hbm.at[pl.ds(base + j * _CHUNK, _CHUNK)],
                ssems[j % _NBUF])

        gh, sh = {}, {}
        for g in range(min(_NBUF - 1, _N_CHUNKS)):
            gh[g] = start_gather(g)
        for j in range(_N_CHUNKS):
            gh[j].wait()
            sh[j] = start_scatter(j)
            g = j + _NBUF - 1
            if g < _N_CHUNKS:
                if g - _NBUF >= 0:
                    sh[g - _NBUF].wait()
                gh[g] = start_gather(g)
        for j in range(max(0, _N_CHUNKS - _NBUF), _N_CHUNKS):
            sh[j].wait()

    return k(table, idx)


def kernel(seq_len, pos_emb):
    positions = jnp.arange(0, SEQ_LEN, dtype=jnp.int32)
    positions = jnp.minimum(positions, jnp.asarray(seq_len, dtype=jnp.int32) - 1)
    idx = positions.reshape(_NW, _N_CHUNKS, _CHUNK)
    return _sc_gather(pos_emb, idx)
